# baseline (speedup 1.0000x reference)
import functools

import jax
import jax.numpy as jnp
from jax import lax
from jax.experimental import pallas as pl
from jax.experimental.pallas import tpu as pltpu
from jax.experimental.pallas import tpu_sc as plsc

_B = 16384
_D = 32
_NC = 2
_NS = 16
_NW = _NC * _NS
_BPW = _B // _NW
_L = 16
_C = 32
_NCHUNK = _BPW // _C


def _build():
    mesh = plsc.VectorSubcoreMesh(core_axis_name="c", subcore_axis_name="s")

    @functools.partial(
        pl.kernel,
        mesh=mesh,
        out_type=jax.ShapeDtypeStruct((_B, _D), jnp.float32),
        scratch_types=[
            pltpu.VMEM((_BPW,), jnp.int32),
            pltpu.VMEM((_C, 8, _D), jnp.float32),
            pltpu.VMEM((_C, 8, _D), jnp.float32),
            pltpu.VMEM((_C, _D), jnp.float32),
            pltpu.VMEM((_C, _D), jnp.float32),
            pltpu.SemaphoreType.DMA,
            pltpu.SemaphoreType.DMA,
        ],
    )
    def gather_kernel(idx_hbm, table_hbm, out_hbm,
                      idx_v, tiles0, tiles1, out0, out1, gsem, osem):
        tiles_b = (tiles0, tiles1)
        out_b = (out0, out1)
        wid = lax.axis_index("s") * _NC + lax.axis_index("c")
        base = wid * _BPW
        pltpu.sync_copy(idx_hbm.at[pl.ds(base, _BPW)], idx_v)
        lanes = lax.iota(jnp.int32, _L)

        def fire(c):
            buf = tiles_b[c % 2]
            descs = []
            for q in range(_C // _L):
                v = idx_v[pl.ds(c * _C + q * _L, _L)]
                for l in range(_L):
                    al = pl.multiple_of(v[l] & jnp.int32(-8), 8)
                    descs.append(
                        pltpu.async_copy(
                            table_hbm.at[pl.ds(al, 8)],
                            buf.at[q * _L + l],
                            gsem,
                        )
                    )
            return descs

        def select(c):
            buf = tiles_b[c % 2]
            flat = buf.reshape(_C * 8, _D)
            ob = out_b[c % 2]
            for q in range(_C // _L):
                v = idx_v[pl.ds(c * _C + q * _L, _L)]
                for l in range(_L):
                    slot = q * _L + l
                    r8s = slot * 8 + (v[l] & 7)
                    ob[slot, pl.ds(0, _L)] = flat[r8s, pl.ds(0, _L)]
                    ob[slot, pl.ds(_L, _L)] = flat[r8s, pl.ds(_L, _L)]

        writes = [None, None]
        descs = fire(0)
        for c in range(_NCHUNK):
            nxt = fire(c + 1) if c + 1 < _NCHUNK else []
            for d in descs:
                d.wait()
            if writes[c % 2] is not None:
                writes[c % 2].wait()
            select(c)
            writes[c % 2] = pltpu.async_copy(
                out_b[c % 2],
                out_hbm.at[pl.ds(base + c * _C, _C)],
                osem,
            )
            descs = nxt
        for w in writes:
            if w is not None:
                w.wait()

    return gather_kernel


def kernel(i, data):
    return _build()(i.astype(jnp.int32), data)
